# baseline (speedup 1.0000x reference)
import math

import jax
import jax.numpy as jnp
from jax.experimental import pallas as pl
from jax.experimental.pallas import tpu as pltpu

_N = 1000
_NP = 1024
_K = 12544
_REP = 1024
_NC = 91
_NCP = 96
_IMG = 800.0
_PRE_NMS = 1000
_DETS = 100
_SCORE_TH = 0.05
_NMS_TH = 0.5
_BB_CLIP = math.log(1000.0 / 16.0)
_KT = 896
_NK = _K // _KT
_NEG = -jnp.inf


def _fused_kernel(x_ref, w6_ref, b6_ref, w7_ref, b7_ref, wct_ref, bcc_ref,
                  wbt_ref, bbr_ref, prop_ref,
                  obx1_ref, oby1_ref, obx2_ref, oby2_ref, os_ref, ol_ref,
                  h1_ref, sc_ref, x1_ref, y1_ref, x2_ref, y2_ref, rm_ref):
    k = pl.program_id(0)
    part = jnp.dot(x_ref[...], w6_ref[...], preferred_element_type=jnp.float32)

    @pl.when(k == 0)
    def _():
        h1_ref[...] = part

    @pl.when(k > 0)
    def _():
        h1_ref[...] = h1_ref[...] + part

    @pl.when(k == _NK - 1)
    def _():
        h1 = jnp.maximum(h1_ref[...] + b6_ref[...], 0.0)
        h2 = jnp.maximum(
            jnp.dot(h1, w7_ref[...], preferred_element_type=jnp.float32)
            + b7_ref[...], 0.0)
        dn = (((1,), (1,)), ((), ()))
        logits = jax.lax.dot_general(
            wct_ref[...], h2, dn,
            preferred_element_type=jnp.float32) + bcc_ref[...]
        deltas = jax.lax.dot_general(
            wbt_ref[...], h2, dn,
            preferred_element_type=jnp.float32) + bbr_ref[...]

        mx = jnp.max(logits, axis=0, keepdims=True)
        e = jnp.exp(logits - mx)
        probs = e / jnp.sum(e, axis=0, keepdims=True)

        pt = prop_ref[...]
        px1, py1, px2, py2 = pt[0:1], pt[1:2], pt[2:3], pt[3:4]
        w = px2 - px1
        ht = py2 - py1
        cx = px1 + 0.5 * w
        cy = py1 + 0.5 * ht
        dx = deltas[0:_NCP] / 10.0
        dy = deltas[_NCP:2 * _NCP] / 10.0
        dw = jnp.minimum(deltas[2 * _NCP:3 * _NCP] / 5.0, _BB_CLIP)
        dh = jnp.minimum(deltas[3 * _NCP:4 * _NCP] / 5.0, _BB_CLIP)
        pcx = dx * w + cx
        pcy = dy * ht + cy
        pw = jnp.exp(dw) * w
        ph = jnp.exp(dh) * ht
        bx1 = jnp.clip(pcx - 0.5 * pw, 0.0, _IMG)
        by1 = jnp.clip(pcy - 0.5 * ph, 0.0, _IMG)
        bx2 = jnp.clip(pcx + 0.5 * pw, 0.0, _IMG)
        by2 = jnp.clip(pcy + 0.5 * ph, 0.0, _IMG)

        riota = jax.lax.broadcasted_iota(jnp.int32, (_NCP, _N), 0)
        base = (riota >= 1) & (riota <= _NC - 1) & (probs > _SCORE_TH)
        mvals = jnp.where(base, probs, 0.0)

        mi = jax.lax.bitcast_convert_type(mvals, jnp.int32)
        kf = float(_PRE_NMS)

        def bis(i, v):
            b = 28 - 2 * i
            c1 = v | jnp.left_shift(jnp.int32(1), b)
            c2 = v | jnp.left_shift(jnp.int32(2), b)
            c3 = v | jnp.left_shift(jnp.int32(3), b)
            n1 = jnp.sum((mi >= c1).astype(jnp.float32))
            n2 = jnp.sum((mi >= c2).astype(jnp.float32))
            n3 = jnp.sum((mi >= c3).astype(jnp.float32))
            return jnp.where(
                n3 >= kf, c3,
                jnp.where(n2 >= kf, c2, jnp.where(n1 >= kf, c1, v)))

        vbits = jax.lax.fori_loop(0, 15, bis, jnp.int32(0))
        vf = jax.lax.bitcast_convert_type(vbits, jnp.float32)
        sc0 = jnp.where(base & (mvals >= vf), probs, _NEG)

        padn = _NP - _N
        scp = jnp.concatenate(
            [sc0, jnp.full((_NCP, padn), _NEG, jnp.float32)], axis=1)
        zpad = jnp.zeros((_NCP, padn), jnp.float32)
        sc_ref[...] = scp.reshape(_NCP, 8, 128)
        x1_ref[...] = jnp.concatenate([bx1, zpad], 1).reshape(_NCP, 8, 128)
        y1_ref[...] = jnp.concatenate([by1, zpad], 1).reshape(_NCP, 8, 128)
        x2_ref[...] = jnp.concatenate([bx2, zpad], 1).reshape(_NCP, 8, 128)
        y2_ref[...] = jnp.concatenate([by2, zpad], 1).reshape(_NCP, 8, 128)
        rm0 = jnp.max(scp, axis=1).reshape(_NCP, 1)
        rm_ref[...] = jnp.concatenate(
            [rm0.T, jnp.full((1, 128 - _NCP), _NEG, jnp.float32)], axis=1)

        iota128 = jax.lax.broadcasted_iota(jnp.int32, (1, 128), 1)

        def body(i, carry):
            rmrow = rm_ref[...]
            m = jnp.max(rmrow)
            r = jnp.min(jnp.where(rmrow == m, iota128, 128))
            sv = sc_ref[pl.ds(r, 1)][0]
            x1v = x1_ref[pl.ds(r, 1)][0]
            y1v = y1_ref[pl.ds(r, 1)][0]
            x2v = x2_ref[pl.ds(r, 1)][0]
            y2v = y2_ref[pl.ds(r, 1)][0]
            sel = sv == m
            cbx1 = jnp.sum(jnp.where(sel, x1v, 0.0))
            cby1 = jnp.sum(jnp.where(sel, y1v, 0.0))
            cbx2 = jnp.sum(jnp.where(sel, x2v, 0.0))
            cby2 = jnp.sum(jnp.where(sel, y2v, 0.0))
            off = r.astype(jnp.float32) * (_IMG + 1.0)
            ox1v = x1v + off
            oy1v = y1v + off
            ox2v = x2v + off
            oy2v = y2v + off
            obx1 = cbx1 + off
            oby1 = cby1 + off
            obx2 = cbx2 + off
            oby2 = cby2 + off
            ix1 = jnp.maximum(obx1, ox1v)
            iy1 = jnp.maximum(oby1, oy1v)
            ix2 = jnp.minimum(obx2, ox2v)
            iy2 = jnp.minimum(oby2, oy2v)
            inter = jnp.maximum(ix2 - ix1, 0.0) * jnp.maximum(iy2 - iy1, 0.0)
            a1 = (obx2 - obx1) * (oby2 - oby1)
            a2 = (ox2v - ox1v) * (oy2v - oy1v)
            iou = inter / (a1 + a2 - inter + 1e-9)
            newsv = jnp.where((iou > _NMS_TH) | sel, _NEG, sv)
            sc_ref[pl.ds(r, 1)] = newsv[None]
            rm_ref[...] = jnp.where(iota128 == r, jnp.max(newsv), rmrow)
            valid = m > -1e30
            obx1_ref[pl.ds(i, 1), :] = jnp.where(valid, cbx1, 0.0).reshape(1, 1)
            oby1_ref[pl.ds(i, 1), :] = jnp.where(valid, cby1, 0.0).reshape(1, 1)
            obx2_ref[pl.ds(i, 1), :] = jnp.where(valid, cbx2, 0.0).reshape(1, 1)
            oby2_ref[pl.ds(i, 1), :] = jnp.where(valid, cby2, 0.0).reshape(1, 1)
            os_ref[pl.ds(i, 1), :] = jnp.where(valid, m, 0.0).reshape(1, 1)
            ol_ref[pl.ds(i, 1), :] = jnp.where(valid, r, 0).reshape(1, 1)
            return carry

        jax.lax.fori_loop(0, _DETS, body, 0)


def kernel(box_features, proposals, W6, b6, W7, b7, Wc, bc, Wb, bb):
    f32 = jnp.float32
    b6r = b6.reshape(1, _REP)
    b7r = b7.reshape(1, _REP)
    pad = _NCP - _NC
    wct = jnp.pad(Wc, ((0, 0), (0, pad))).T
    bcc = jnp.pad(bc, (0, pad), constant_values=-1e30).reshape(_NCP, 1)
    wbt = jnp.pad(
        jnp.transpose(Wb.reshape(_REP, _NC, 4), (2, 1, 0)),
        ((0, 0), (0, pad), (0, 0))).reshape(4 * _NCP, _REP)
    bbr = jnp.pad(bb.reshape(_NC, 4).T, ((0, 0), (0, pad))).reshape(4 * _NCP, 1)
    propt = jnp.pad(proposals.T, ((0, 4), (0, 0)))

    cidx = lambda k: (0, 0)
    outs = pl.pallas_call(
        _fused_kernel,
        grid=(_NK,),
        in_specs=[
            pl.BlockSpec((_N, _KT), lambda k: (0, k)),
            pl.BlockSpec((_KT, _REP), lambda k: (k, 0)),
            pl.BlockSpec((1, _REP), cidx),
            pl.BlockSpec((_REP, _REP), cidx),
            pl.BlockSpec((1, _REP), cidx),
            pl.BlockSpec((_NCP, _REP), cidx),
            pl.BlockSpec((_NCP, 1), cidx),
            pl.BlockSpec((4 * _NCP, _REP), cidx),
            pl.BlockSpec((4 * _NCP, 1), cidx),
            pl.BlockSpec((8, _N), cidx),
        ],
        out_specs=[
            pl.BlockSpec((_DETS, 1), cidx),
            pl.BlockSpec((_DETS, 1), cidx),
            pl.BlockSpec((_DETS, 1), cidx),
            pl.BlockSpec((_DETS, 1), cidx),
            pl.BlockSpec((_DETS, 1), cidx),
            pl.BlockSpec((_DETS, 1), cidx),
        ],
        out_shape=[
            jax.ShapeDtypeStruct((_DETS, 1), f32),
            jax.ShapeDtypeStruct((_DETS, 1), f32),
            jax.ShapeDtypeStruct((_DETS, 1), f32),
            jax.ShapeDtypeStruct((_DETS, 1), f32),
            jax.ShapeDtypeStruct((_DETS, 1), f32),
            jax.ShapeDtypeStruct((_DETS, 1), jnp.int32),
        ],
        scratch_shapes=[
            pltpu.VMEM((_N, _REP), f32),
            pltpu.VMEM((_NCP, 8, 128), f32),
            pltpu.VMEM((_NCP, 8, 128), f32),
            pltpu.VMEM((_NCP, 8, 128), f32),
            pltpu.VMEM((_NCP, 8, 128), f32),
            pltpu.VMEM((_NCP, 8, 128), f32),
            pltpu.VMEM((1, 128), f32),
        ],
    )(box_features, W6, b6r, W7, b7r, wct, bcc, wbt, bbr, propt)
    obx1, oby1, obx2, oby2, osc, olb = outs
    out_boxes = jnp.concatenate([obx1, oby1, obx2, oby2], axis=1)
    return out_boxes, osc.reshape(_DETS), olb.reshape(_DETS)
